# Initial kernel scaffold; baseline (speedup 1.0000x reference)
import jax
import jax.numpy as jnp
from jax.experimental import pallas as pl


def kernel(x, edge_index, edge_attr, W1, b1, W2, b2):
    raise NotImplementedError("write your pallas kernel here")


# speedup vs baseline: 11.1134x; 11.1134x over previous
import functools

import jax
import jax.numpy as jnp
from jax import lax
from jax.experimental import pallas as pl
from jax.experimental.pallas import tpu as pltpu
from jax.experimental.pallas import tpu_sc as plsc

N = 10000
E = 320000
D_IN = 128
D_HID = 64
D_OUT = 128

NC = 2
NS = 16
NW = NC * NS
N_PAD = 10240
ROWS_PER_TILE = N_PAD // NS

CH = 80
EA = E + N
EPW_A = 10320
EA_PAD = EPW_A * NW
NCH_A = EPW_A // CH

EPW_D = E // NW
NCH_D = EPW_D // CH

BM = 1024

_mesh = functools.partial(
    plsc.VectorSubcoreMesh, core_axis_name="c", subcore_axis_name="s")


@functools.partial(
    pl.kernel,
    out_type=jax.ShapeDtypeStruct((NC, N_PAD), jnp.float32),
    mesh=_mesh(),
    scratch_types=[
        pltpu.VMEM((CH,), jnp.int32),
        pltpu.VMEM((CH,), jnp.float32),
        pltpu.VMEM_SHARED((N_PAD,), jnp.float32),
        pltpu.SemaphoreType.DMA,
    ],
)
def _sc_degree(dst_hbm, ew_hbm, z_hbm, out_hbm, dst_v, ew_v, acc_sh, sem):
    cid = lax.axis_index("c")
    sid = lax.axis_index("s")
    wid = sid * NC + cid
    seg = pl.ds(sid * ROWS_PER_TILE, ROWS_PER_TILE)
    pltpu.sync_copy(z_hbm.at[seg], acc_sh.at[seg])
    plsc.subcore_barrier()

    def chunk(i, carry):
        base = pl.multiple_of(wid * EPW_D + i * CH, CH)
        d1 = pltpu.async_copy(dst_hbm.at[pl.ds(base, CH)], dst_v, sem)
        d2 = pltpu.async_copy(ew_hbm.at[pl.ds(base, CH)], ew_v, sem)
        d1.wait()
        d2.wait()
        pltpu.sync_copy(ew_v, acc_sh.at[dst_v], add=True)
        return carry

    lax.fori_loop(0, NCH_D, chunk, 0)
    plsc.subcore_barrier()
    pltpu.sync_copy(acc_sh.at[seg], out_hbm.at[cid, seg])


def _make_sc_layer(d_feat, first_layer):
    out_acc = jax.ShapeDtypeStruct((NC, N_PAD, d_feat), jnp.float32)
    scratch = [
        pltpu.VMEM((CH,), jnp.int32),
        pltpu.VMEM((CH,), jnp.int32),
        pltpu.VMEM((CH,), jnp.float32),
        pltpu.VMEM((CH, d_feat), jnp.float32),
        pltpu.VMEM_SHARED((N_PAD, d_feat), jnp.float32),
        pltpu.SemaphoreType.DMA,
    ]
    if first_layer:
        out_type = (out_acc, jax.ShapeDtypeStruct((EA_PAD,), jnp.float32))
        scratch = [pltpu.VMEM((CH,), jnp.float32),
                   pltpu.VMEM((CH,), jnp.float32),
                   pltpu.VMEM((CH,), jnp.float32)] + scratch
    else:
        out_type = out_acc

    def body(*refs):
        if first_layer:
            (src_hbm, dst_hbm, ew_hbm, dinv_hbm, h_hbm, z_hbm,
             acc_out, norm_out,
             ew_v, dvs_v, dvd_v, src_v, dst_v, norm_v, rows_v,
             acc_sh, sem) = refs
        else:
            (src_hbm, dst_hbm, norm_hbm, h_hbm, z_hbm,
             acc_out,
             src_v, dst_v, norm_v, rows_v,
             acc_sh, sem) = refs
        cid = lax.axis_index("c")
        sid = lax.axis_index("s")
        wid = sid * NC + cid
        seg = pl.ds(sid * ROWS_PER_TILE, ROWS_PER_TILE)
        pltpu.sync_copy(z_hbm.at[seg], acc_sh.at[seg])
        plsc.subcore_barrier()

        def chunk(i, carry):
            base = pl.multiple_of(wid * EPW_A + i * CH, CH)
            esl = pl.ds(base, CH)
            d1 = pltpu.async_copy(src_hbm.at[esl], src_v, sem)
            d2 = pltpu.async_copy(dst_hbm.at[esl], dst_v, sem)
            if first_layer:
                d3 = pltpu.async_copy(ew_hbm.at[esl], ew_v, sem)
            else:
                d3 = pltpu.async_copy(norm_hbm.at[esl], norm_v, sem)
            d1.wait()
            d2.wait()
            d3.wait()
            g1 = pltpu.async_copy(h_hbm.at[src_v], rows_v, sem)
            if first_layer:
                g2 = pltpu.async_copy(dinv_hbm.at[src_v], dvs_v, sem)
                g3 = pltpu.async_copy(dinv_hbm.at[dst_v], dvd_v, sem)
                g2.wait()
                g3.wait()
            g1.wait()
            if first_layer:
                for j in range(CH // 16):
                    sl = pl.ds(j * 16, 16)
                    norm_v[sl] = dvs_v[sl] * ew_v[sl] * dvd_v[sl]
                pltpu.sync_copy(norm_v, norm_out.at[esl])

            def row(r, rcarry):
                s = plsc.load_gather(
                    norm_v, [jnp.full((16,), r, dtype=jnp.int32)])
                for j in range(d_feat // 16):
                    fsl = pl.ds(j * 16, 16)
                    rows_v[r, fsl] = rows_v[r, fsl] * s
                return rcarry

            lax.fori_loop(0, CH, row, 0)
            pltpu.sync_copy(rows_v, acc_sh.at[dst_v], add=True)
            return carry

        lax.fori_loop(0, NCH_A, chunk, 0)
        plsc.subcore_barrier()
        pltpu.sync_copy(acc_sh.at[seg], acc_out.at[cid, seg])

    return pl.kernel(
        body, out_type=out_type, mesh=_mesh(), scratch_types=scratch,
        compiler_params=pltpu.CompilerParams(use_tc_tiling_on_sc=False,
                                             needs_layout_passes=False))


_sc_layer1 = _make_sc_layer(D_HID, first_layer=True)
_sc_layer2 = _make_sc_layer(D_OUT, first_layer=False)


def _tc_dinv(p0, p1):
    def body(p0_ref, p1_ref, out_ref):
        deg = 1.0 + p0_ref[...] + p1_ref[...]
        out_ref[...] = lax.rsqrt(deg)

    shp = jax.ShapeDtypeStruct(p0.shape, jnp.float32)
    return pl.pallas_call(body, out_shape=shp)(p0, p1)


def _tc_matmul(xp, W):
    def body(x_ref, w_ref, out_ref):
        out_ref[...] = jnp.dot(x_ref[...], w_ref[...],
                               preferred_element_type=jnp.float32)

    m, k = xp.shape
    n = W.shape[1]
    return pl.pallas_call(
        body,
        grid=(m // BM,),
        in_specs=[pl.BlockSpec((BM, k), lambda i: (i, 0)),
                  pl.BlockSpec((k, n), lambda i: (0, 0))],
        out_specs=pl.BlockSpec((BM, n), lambda i: (i, 0)),
        out_shape=jax.ShapeDtypeStruct((m, n), jnp.float32),
    )(xp, W)


def _tc_combine_matmul(a0, a1, bb, W):
    def body(a0_ref, a1_ref, b_ref, w_ref, out_ref):
        o = a0_ref[...] + a1_ref[...] + b_ref[0:1, :]
        out_ref[...] = jnp.dot(o, w_ref[...],
                               preferred_element_type=jnp.float32)

    m, k = a0.shape
    n = W.shape[1]
    return pl.pallas_call(
        body,
        grid=(m // BM,),
        in_specs=[pl.BlockSpec((BM, k), lambda i: (i, 0)),
                  pl.BlockSpec((BM, k), lambda i: (i, 0)),
                  pl.BlockSpec((8, k), lambda i: (0, 0)),
                  pl.BlockSpec((k, n), lambda i: (0, 0))],
        out_specs=pl.BlockSpec((BM, n), lambda i: (i, 0)),
        out_shape=jax.ShapeDtypeStruct((m, n), jnp.float32),
    )(a0, a1, bb, W)


def _tc_final(a0, a1, bb):
    def body(a0_ref, a1_ref, b_ref, out_ref):
        z = a0_ref[...] + a1_ref[...] + b_ref[0:1, :]
        z = z - jnp.max(z, axis=1, keepdims=True)
        ez = jnp.exp(z)
        out_ref[...] = ez / jnp.sum(ez, axis=1, keepdims=True)

    m, n = a0.shape
    return pl.pallas_call(
        body,
        grid=(m // BM,),
        in_specs=[pl.BlockSpec((BM, n), lambda i: (i, 0)),
                  pl.BlockSpec((BM, n), lambda i: (i, 0)),
                  pl.BlockSpec((8, n), lambda i: (0, 0))],
        out_specs=pl.BlockSpec((BM, n), lambda i: (i, 0)),
        out_shape=jax.ShapeDtypeStruct((m, n), jnp.float32),
    )(a0, a1, bb)


def kernel(x, edge_index, edge_attr, W1, b1, W2, b2):
    loop = jnp.arange(N, dtype=jnp.int32)
    pad_e = EA_PAD - EA
    src_all = jnp.concatenate(
        [edge_index[0], loop, jnp.zeros((pad_e,), jnp.int32)])
    dst_all = jnp.concatenate(
        [edge_index[1], loop, jnp.zeros((pad_e,), jnp.int32)])
    ew_all = jnp.concatenate(
        [edge_attr, jnp.ones((N,), jnp.float32),
         jnp.zeros((pad_e,), jnp.float32)])

    x_pad = jnp.pad(x, ((0, N_PAD - N), (0, 0)))
    z1 = jnp.zeros((N_PAD,), jnp.float32)
    z64 = jnp.zeros((N_PAD, D_HID), jnp.float32)
    z128 = jnp.zeros((N_PAD, D_OUT), jnp.float32)
    b1r = jnp.broadcast_to(b1, (8, D_HID))
    b2r = jnp.broadcast_to(b2, (8, D_OUT))

    deg_part = _sc_degree(edge_index[1], edge_attr, z1)
    dinv = _tc_dinv(deg_part[0].reshape(N_PAD // 128, 128),
                    deg_part[1].reshape(N_PAD // 128, 128)).reshape(N_PAD)

    h1 = _tc_matmul(x_pad, W1)
    acc1, norm = _sc_layer1(src_all, dst_all, ew_all, dinv, h1, z64)
    h2 = _tc_combine_matmul(acc1[0], acc1[1], b1r, W2)
    acc2 = _sc_layer2(src_all, dst_all, norm, h2, z128)
    out = _tc_final(acc2[0], acc2[1], b2r)
    return out[:N]


# speedup vs baseline: 17.3325x; 1.5596x over previous
import functools

import jax
import jax.numpy as jnp
from jax import lax
from jax.experimental import pallas as pl
from jax.experimental.pallas import tpu as pltpu
from jax.experimental.pallas import tpu_sc as plsc

N = 10000
E = 320000
D_IN = 128
D_HID = 64
D_OUT = 128

NC = 2
NS = 16
NW = NC * NS
N_PAD = 10112
ROWS_PER_TILE = N_PAD // NS
NT = N // NS

CH = 96
U = 4
EA = E + N
NCH_A = 108
EPW_A = NCH_A * CH
EA_PAD = EPW_A * NW

UD = 5
CHD = 80
EPW_D = E // NW
NCH_D = EPW_D // CHD

BM = 2000

_mesh = functools.partial(
    plsc.VectorSubcoreMesh, core_axis_name="c", subcore_axis_name="s")
_params = pltpu.CompilerParams(use_tc_tiling_on_sc=False,
                               needs_layout_passes=False)


@functools.partial(
    pl.kernel,
    out_type=jax.ShapeDtypeStruct((NC, N_PAD), jnp.float32),
    mesh=_mesh(),
    scratch_types=[
        pltpu.VMEM((UD, CHD), jnp.int32),
        pltpu.VMEM((UD, CHD), jnp.float32),
        pltpu.VMEM_SHARED((N_PAD,), jnp.float32),
    ] + [pltpu.SemaphoreType.DMA] * 2,
    compiler_params=_params,
)
def _sc_degree(dst_hbm, ew_hbm, z_hbm, out_hbm, dst_v, ew_v, acc_sh, *sems):
    semL = [sems[0]] * UD
    semS = [sems[1]] * UD
    cid = lax.axis_index("c")
    sid = lax.axis_index("s")
    wid = sid * NC + cid
    seg = pl.ds(sid * ROWS_PER_TILE, ROWS_PER_TILE)
    pltpu.sync_copy(z_hbm.at[seg], acc_sh.at[seg])
    plsc.subcore_barrier()

    def iteration(g, carry):
        i0 = g * UD
        descs = []
        for k in range(UD):
            base = pl.multiple_of(wid * EPW_D + (i0 + k) * CHD, CHD)
            esl = pl.ds(base, CHD)
            descs.append(
                (pltpu.async_copy(dst_hbm.at[esl], dst_v.at[k], semL[k]),
                 pltpu.async_copy(ew_hbm.at[esl], ew_v.at[k], semL[k])))
        sdescs = []
        for k in range(UD):
            descs[k][0].wait()
            descs[k][1].wait()
            sdescs.append(
                pltpu.async_copy(ew_v.at[k], acc_sh.at[dst_v.at[k]],
                                 semS[k], add=True))
        for k in range(UD):
            sdescs[k].wait()
        return carry

    lax.fori_loop(0, NCH_D // UD, iteration, 0)
    plsc.subcore_barrier()
    pltpu.sync_copy(acc_sh.at[seg], out_hbm.at[cid, seg])


def _make_sc_layer(d_feat, first_layer):
    out_acc = jax.ShapeDtypeStruct((NC, N, d_feat), jnp.float32)
    scratch = [
        pltpu.VMEM((U, CH), jnp.int32),
        pltpu.VMEM((U, CH), jnp.int32),
        pltpu.VMEM((U, CH), jnp.float32),
        pltpu.VMEM((U, CH, d_feat), jnp.float32),
        pltpu.VMEM_SHARED((N, d_feat), jnp.float32),
    ]
    nsem = 3 + (1 if first_layer else 0)
    scratch += [pltpu.SemaphoreType.DMA] * nsem
    if first_layer:
        out_type = (out_acc, jax.ShapeDtypeStruct((EA_PAD,), jnp.float32))
        scratch = [pltpu.VMEM((U, CH), jnp.float32),
                   pltpu.VMEM((U, CH), jnp.float32),
                   pltpu.VMEM((U, CH), jnp.float32)] + scratch
    else:
        out_type = out_acc

    def body(*refs):
        if first_layer:
            (src_hbm, dst_hbm, ew_hbm, dinv_hbm, h_hbm, z_hbm,
             acc_out, norm_out,
             ew_v, dvs_v, dvd_v, src_v, dst_v, norm_v, rows_v,
             acc_sh, *sems) = refs
        else:
            (src_hbm, dst_hbm, norm_hbm, h_hbm, z_hbm,
             acc_out,
             src_v, dst_v, norm_v, rows_v,
             acc_sh, *sems) = refs
        semL = [sems[0]] * U
        semG = [sems[1]] * U
        semS = [sems[2]] * U
        semN = [sems[3]] * U if first_layer else None
        cid = lax.axis_index("c")
        sid = lax.axis_index("s")
        wid = sid * NC + cid
        seg = pl.ds(sid * NT, NT)
        pltpu.sync_copy(z_hbm.at[seg], acc_sh.at[seg])
        plsc.subcore_barrier()

        def iteration(g, carry):
            i0 = g * U
            bases = []
            ldescs = []
            for k in range(U):
                base = pl.multiple_of(wid * EPW_A + (i0 + k) * CH, CH)
                esl = pl.ds(base, CH)
                bases.append(esl)
                d = [pltpu.async_copy(src_hbm.at[esl], src_v.at[k], semL[k]),
                     pltpu.async_copy(dst_hbm.at[esl], dst_v.at[k], semL[k])]
                if first_layer:
                    d.append(pltpu.async_copy(ew_hbm.at[esl], ew_v.at[k],
                                              semL[k]))
                else:
                    d.append(pltpu.async_copy(norm_hbm.at[esl], norm_v.at[k],
                                              semL[k]))
                ldescs.append(d)
            gdescs = []
            for k in range(U):
                for d in ldescs[k]:
                    d.wait()
                g_ = [pltpu.async_copy(h_hbm.at[src_v.at[k]], rows_v.at[k],
                                       semG[k])]
                if first_layer:
                    g_.append(pltpu.async_copy(dinv_hbm.at[src_v.at[k]],
                                               dvs_v.at[k], semG[k]))
                    g_.append(pltpu.async_copy(dinv_hbm.at[dst_v.at[k]],
                                               dvd_v.at[k], semG[k]))
                gdescs.append(g_)
            sdescs = []
            ndescs = []
            for k in range(U):
                for d in gdescs[k]:
                    d.wait()
                if first_layer:
                    for j in range(CH // 16):
                        sl = pl.ds(j * 16, 16)
                        norm_v[k, sl] = dvs_v[k, sl] * ew_v[k, sl] * dvd_v[k, sl]
                    ndescs.append(pltpu.async_copy(norm_v.at[k],
                                                   norm_out.at[bases[k]],
                                                   semN[k]))

                def row(r, rcarry):
                    s = plsc.load_gather(
                        norm_v, [jnp.full((16,), k, dtype=jnp.int32),
                                 jnp.full((16,), r, dtype=jnp.int32)])
                    for j in range(d_feat // 16):
                        fsl = pl.ds(j * 16, 16)
                        rows_v[k, r, fsl] = rows_v[k, r, fsl] * s
                    return rcarry

                lax.fori_loop(0, CH, row, 0)
                sdescs.append(
                    pltpu.async_copy(rows_v.at[k], acc_sh.at[dst_v.at[k]],
                                     semS[k], add=True))
            for k in range(U):
                sdescs[k].wait()
                if first_layer:
                    ndescs[k].wait()
            return carry

        lax.fori_loop(0, NCH_A // U, iteration, 0)
        plsc.subcore_barrier()
        pltpu.sync_copy(acc_sh.at[seg], acc_out.at[cid, seg])

    return pl.kernel(body, out_type=out_type, mesh=_mesh(),
                     scratch_types=scratch, compiler_params=_params)


_sc_layer1 = _make_sc_layer(D_HID, first_layer=True)
_sc_layer2 = _make_sc_layer(D_OUT, first_layer=False)


def _tc_dinv(p0, p1):
    def body(p0_ref, p1_ref, out_ref):
        deg = 1.0 + p0_ref[...] + p1_ref[...]
        out_ref[...] = lax.rsqrt(deg)

    shp = jax.ShapeDtypeStruct(p0.shape, jnp.float32)
    return pl.pallas_call(body, out_shape=shp)(p0, p1)


def _tc_matmul(xp, W):
    def body(x_ref, w_ref, out_ref):
        out_ref[...] = jnp.dot(x_ref[...], w_ref[...],
                               preferred_element_type=jnp.float32)

    m, k = xp.shape
    n = W.shape[1]
    return pl.pallas_call(
        body,
        grid=(m // BM,),
        in_specs=[pl.BlockSpec((BM, k), lambda i: (i, 0)),
                  pl.BlockSpec((k, n), lambda i: (0, 0))],
        out_specs=pl.BlockSpec((BM, n), lambda i: (i, 0)),
        out_shape=jax.ShapeDtypeStruct((m, n), jnp.float32),
    )(xp, W)


def _tc_combine_matmul(a0, a1, bb, W):
    def body(a0_ref, a1_ref, b_ref, w_ref, out_ref):
        o = a0_ref[...] + a1_ref[...] + b_ref[0:1, :]
        out_ref[...] = jnp.dot(o, w_ref[...],
                               preferred_element_type=jnp.float32)

    m, k = a0.shape
    n = W.shape[1]
    return pl.pallas_call(
        body,
        grid=(m // BM,),
        in_specs=[pl.BlockSpec((BM, k), lambda i: (i, 0)),
                  pl.BlockSpec((BM, k), lambda i: (i, 0)),
                  pl.BlockSpec((8, k), lambda i: (0, 0)),
                  pl.BlockSpec((k, n), lambda i: (0, 0))],
        out_specs=pl.BlockSpec((BM, n), lambda i: (i, 0)),
        out_shape=jax.ShapeDtypeStruct((m, n), jnp.float32),
    )(a0, a1, bb, W)


def _tc_final(a0, a1, bb):
    def body(a0_ref, a1_ref, b_ref, out_ref):
        z = a0_ref[...] + a1_ref[...] + b_ref[0:1, :]
        z = z - jnp.max(z, axis=1, keepdims=True)
        ez = jnp.exp(z)
        out_ref[...] = ez / jnp.sum(ez, axis=1, keepdims=True)

    m, n = a0.shape
    return pl.pallas_call(
        body,
        grid=(m // BM,),
        in_specs=[pl.BlockSpec((BM, n), lambda i: (i, 0)),
                  pl.BlockSpec((BM, n), lambda i: (i, 0)),
                  pl.BlockSpec((8, n), lambda i: (0, 0))],
        out_specs=pl.BlockSpec((BM, n), lambda i: (i, 0)),
        out_shape=jax.ShapeDtypeStruct((m, n), jnp.float32),
    )(a0, a1, bb)


def kernel(x, edge_index, edge_attr, W1, b1, W2, b2):
    loop = jnp.arange(N, dtype=jnp.int32)
    pad_e = EA_PAD - EA
    src_all = jnp.concatenate(
        [edge_index[0], loop, jnp.zeros((pad_e,), jnp.int32)])
    dst_all = jnp.concatenate(
        [edge_index[1], loop, jnp.zeros((pad_e,), jnp.int32)])
    ew_all = jnp.concatenate(
        [edge_attr, jnp.ones((N,), jnp.float32),
         jnp.zeros((pad_e,), jnp.float32)])

    z1 = jnp.zeros((N_PAD,), jnp.float32)
    z64 = jnp.zeros((N, D_HID), jnp.float32)
    z128 = jnp.zeros((N, D_OUT), jnp.float32)
    b1r = jnp.broadcast_to(b1, (8, D_HID))
    b2r = jnp.broadcast_to(b2, (8, D_OUT))

    deg_part = _sc_degree(edge_index[1], edge_attr, z1)
    dinv = _tc_dinv(deg_part[0].reshape(N_PAD // 128, 128),
                    deg_part[1].reshape(N_PAD // 128, 128)).reshape(N_PAD)

    h1 = _tc_matmul(x, W1)
    acc1, norm = _sc_layer1(src_all, dst_all, ew_all, dinv, h1, z64)
    h2 = _tc_combine_matmul(acc1[0], acc1[1], b1r, W2)
    acc2 = _sc_layer2(src_all, dst_all, norm, h2, z128)
    out = _tc_final(acc2[0], acc2[1], b2r)
    return out


# speedup vs baseline: 17.6441x; 1.0180x over previous
import functools

import jax
import jax.numpy as jnp
from jax import lax
from jax.experimental import pallas as pl
from jax.experimental.pallas import tpu as pltpu
from jax.experimental.pallas import tpu_sc as plsc

N = 10000
E = 320000
D_IN = 128
D_HID = 64
D_OUT = 128

NC = 2
NS = 16
NW = NC * NS
N_PAD = 10112
ROWS_PER_TILE = N_PAD // NS
NT = N // NS

CH = 96
U = 4
RU = 8
EA = E + N
NCH_A = 108
EPW_A = NCH_A * CH
EA_PAD = EPW_A * NW

UD = 5
CHD = 80
EPW_D = E // NW
NCH_D = EPW_D // CHD

BM = 2000

_mesh = functools.partial(
    plsc.VectorSubcoreMesh, core_axis_name="c", subcore_axis_name="s")
_params = pltpu.CompilerParams(use_tc_tiling_on_sc=False,
                               needs_layout_passes=False)


@functools.partial(
    pl.kernel,
    out_type=jax.ShapeDtypeStruct((NC, N_PAD), jnp.float32),
    mesh=_mesh(),
    scratch_types=[
        pltpu.VMEM((UD, CHD), jnp.int32),
        pltpu.VMEM((UD, CHD), jnp.float32),
        pltpu.VMEM_SHARED((N_PAD,), jnp.float32),
    ] + [pltpu.SemaphoreType.DMA] * 2,
    compiler_params=_params,
)
def _sc_degree(dst_hbm, ew_hbm, z_hbm, out_hbm, dst_v, ew_v, acc_sh, *sems):
    semL = [sems[0]] * UD
    semS = [sems[1]] * UD
    cid = lax.axis_index("c")
    sid = lax.axis_index("s")
    wid = sid * NC + cid
    seg = pl.ds(sid * ROWS_PER_TILE, ROWS_PER_TILE)
    pltpu.sync_copy(z_hbm.at[seg], acc_sh.at[seg])
    plsc.subcore_barrier()

    def iteration(g, carry):
        i0 = g * UD
        descs = []
        for k in range(UD):
            base = pl.multiple_of(wid * EPW_D + (i0 + k) * CHD, CHD)
            esl = pl.ds(base, CHD)
            descs.append(
                (pltpu.async_copy(dst_hbm.at[esl], dst_v.at[k], semL[k]),
                 pltpu.async_copy(ew_hbm.at[esl], ew_v.at[k], semL[k])))
        sdescs = []
        for k in range(UD):
            descs[k][0].wait()
            descs[k][1].wait()
            sdescs.append(
                pltpu.async_copy(ew_v.at[k], acc_sh.at[dst_v.at[k]],
                                 semS[k], add=True))
        for k in range(UD):
            sdescs[k].wait()
        return carry

    lax.fori_loop(0, NCH_D // UD, iteration, 0)
    plsc.subcore_barrier()
    pltpu.sync_copy(acc_sh.at[seg], out_hbm.at[cid, seg])


def _make_sc_layer(d_feat, first_layer):
    out_acc = jax.ShapeDtypeStruct((NC, N, d_feat), jnp.float32)
    scratch = [
        pltpu.VMEM((U, CH), jnp.int32),
        pltpu.VMEM((U, CH), jnp.int32),
        pltpu.VMEM((U, CH), jnp.float32),
        pltpu.VMEM((U, CH, d_feat), jnp.float32),
        pltpu.VMEM_SHARED((N, d_feat), jnp.float32),
    ]
    nsem = 3 + (1 if first_layer else 0)
    scratch += [pltpu.SemaphoreType.DMA] * nsem
    if first_layer:
        out_type = (out_acc, jax.ShapeDtypeStruct((EA_PAD,), jnp.float32))
        scratch = [pltpu.VMEM((U, CH), jnp.float32),
                   pltpu.VMEM((U, CH), jnp.float32),
                   pltpu.VMEM((U, CH), jnp.float32)] + scratch
    else:
        out_type = out_acc

    def body(*refs):
        if first_layer:
            (src_hbm, dst_hbm, ew_hbm, dinv_hbm, h_hbm, z_hbm,
             acc_out, norm_out,
             ew_v, dvs_v, dvd_v, src_v, dst_v, norm_v, rows_v,
             acc_sh, *sems) = refs
        else:
            (src_hbm, dst_hbm, norm_hbm, h_hbm, z_hbm,
             acc_out,
             src_v, dst_v, norm_v, rows_v,
             acc_sh, *sems) = refs
        semL = [sems[0]] * U
        semG = [sems[1]] * U
        semS = [sems[2]] * U
        semN = [sems[3]] * U if first_layer else None
        cid = lax.axis_index("c")
        sid = lax.axis_index("s")
        wid = sid * NC + cid
        seg = pl.ds(sid * NT, NT)
        pltpu.sync_copy(z_hbm.at[seg], acc_sh.at[seg])
        plsc.subcore_barrier()

        def iteration(g, carry):
            i0 = g * U
            bases = []
            ldescs = []
            for k in range(U):
                base = pl.multiple_of(wid * EPW_A + (i0 + k) * CH, CH)
                esl = pl.ds(base, CH)
                bases.append(esl)
                d = [pltpu.async_copy(src_hbm.at[esl], src_v.at[k], semL[k]),
                     pltpu.async_copy(dst_hbm.at[esl], dst_v.at[k], semL[k])]
                if first_layer:
                    d.append(pltpu.async_copy(ew_hbm.at[esl], ew_v.at[k],
                                              semL[k]))
                else:
                    d.append(pltpu.async_copy(norm_hbm.at[esl], norm_v.at[k],
                                              semL[k]))
                ldescs.append(d)
            gdescs = []
            for k in range(U):
                for d in ldescs[k]:
                    d.wait()
                g_ = [pltpu.async_copy(h_hbm.at[src_v.at[k]], rows_v.at[k],
                                       semG[k])]
                if first_layer:
                    g_.append(pltpu.async_copy(dinv_hbm.at[src_v.at[k]],
                                               dvs_v.at[k], semG[k]))
                    g_.append(pltpu.async_copy(dinv_hbm.at[dst_v.at[k]],
                                               dvd_v.at[k], semG[k]))
                gdescs.append(g_)
            sdescs = []
            ndescs = []
            for k in range(U):
                for d in gdescs[k]:
                    d.wait()
                if first_layer:
                    for j in range(CH // 16):
                        sl = pl.ds(j * 16, 16)
                        norm_v[k, sl] = dvs_v[k, sl] * ew_v[k, sl] * dvd_v[k, sl]
                    ndescs.append(pltpu.async_copy(norm_v.at[k],
                                                   norm_out.at[bases[k]],
                                                   semN[k]))

                def row(r0, rcarry):
                    for rr in range(RU):
                        r = r0 * RU + rr
                        s = plsc.load_gather(
                            norm_v, [jnp.full((16,), k, dtype=jnp.int32),
                                     jnp.full((16,), r, dtype=jnp.int32)])
                        for j in range(d_feat // 16):
                            fsl = pl.ds(j * 16, 16)
                            rows_v[k, r, fsl] = rows_v[k, r, fsl] * s
                    return rcarry

                lax.fori_loop(0, CH // RU, row, 0)
                sdescs.append(
                    pltpu.async_copy(rows_v.at[k], acc_sh.at[dst_v.at[k]],
                                     semS[k], add=True))
            for k in range(U):
                sdescs[k].wait()
                if first_layer:
                    ndescs[k].wait()
            return carry

        lax.fori_loop(0, NCH_A // U, iteration, 0)
        plsc.subcore_barrier()
        pltpu.sync_copy(acc_sh.at[seg], acc_out.at[cid, seg])

    return pl.kernel(body, out_type=out_type, mesh=_mesh(),
                     scratch_types=scratch, compiler_params=_params)


_sc_layer1 = _make_sc_layer(D_HID, first_layer=True)
_sc_layer2 = _make_sc_layer(D_OUT, first_layer=False)


def _tc_dinv(p0, p1):
    def body(p0_ref, p1_ref, out_ref):
        deg = 1.0 + p0_ref[...] + p1_ref[...]
        out_ref[...] = lax.rsqrt(deg)

    shp = jax.ShapeDtypeStruct(p0.shape, jnp.float32)
    return pl.pallas_call(body, out_shape=shp)(p0, p1)


def _tc_matmul(xp, W):
    def body(x_ref, w_ref, out_ref):
        out_ref[...] = jnp.dot(x_ref[...], w_ref[...],
                               preferred_element_type=jnp.float32)

    m, k = xp.shape
    n = W.shape[1]
    return pl.pallas_call(
        body,
        grid=(m // BM,),
        in_specs=[pl.BlockSpec((BM, k), lambda i: (i, 0)),
                  pl.BlockSpec((k, n), lambda i: (0, 0))],
        out_specs=pl.BlockSpec((BM, n), lambda i: (i, 0)),
        out_shape=jax.ShapeDtypeStruct((m, n), jnp.float32),
    )(xp, W)


def _tc_combine_matmul(a0, a1, bb, W):
    def body(a0_ref, a1_ref, b_ref, w_ref, out_ref):
        o = a0_ref[...] + a1_ref[...] + b_ref[0:1, :]
        out_ref[...] = jnp.dot(o, w_ref[...],
                               preferred_element_type=jnp.float32)

    m, k = a0.shape
    n = W.shape[1]
    return pl.pallas_call(
        body,
        grid=(m // BM,),
        in_specs=[pl.BlockSpec((BM, k), lambda i: (i, 0)),
                  pl.BlockSpec((BM, k), lambda i: (i, 0)),
                  pl.BlockSpec((8, k), lambda i: (0, 0)),
                  pl.BlockSpec((k, n), lambda i: (0, 0))],
        out_specs=pl.BlockSpec((BM, n), lambda i: (i, 0)),
        out_shape=jax.ShapeDtypeStruct((m, n), jnp.float32),
    )(a0, a1, bb, W)


def _tc_final(a0, a1, bb):
    def body(a0_ref, a1_ref, b_ref, out_ref):
        z = a0_ref[...] + a1_ref[...] + b_ref[0:1, :]
        z = z - jnp.max(z, axis=1, keepdims=True)
        ez = jnp.exp(z)
        out_ref[...] = ez / jnp.sum(ez, axis=1, keepdims=True)

    m, n = a0.shape
    return pl.pallas_call(
        body,
        grid=(m // BM,),
        in_specs=[pl.BlockSpec((BM, n), lambda i: (i, 0)),
                  pl.BlockSpec((BM, n), lambda i: (i, 0)),
                  pl.BlockSpec((8, n), lambda i: (0, 0))],
        out_specs=pl.BlockSpec((BM, n), lambda i: (i, 0)),
        out_shape=jax.ShapeDtypeStruct((m, n), jnp.float32),
    )(a0, a1, bb)


def kernel(x, edge_index, edge_attr, W1, b1, W2, b2):
    loop = jnp.arange(N, dtype=jnp.int32)
    pad_e = EA_PAD - EA
    src_all = jnp.concatenate(
        [edge_index[0], loop, jnp.zeros((pad_e,), jnp.int32)])
    dst_all = jnp.concatenate(
        [edge_index[1], loop, jnp.zeros((pad_e,), jnp.int32)])
    ew_all = jnp.concatenate(
        [edge_attr, jnp.ones((N,), jnp.float32),
         jnp.zeros((pad_e,), jnp.float32)])

    z1 = jnp.zeros((N_PAD,), jnp.float32)
    z64 = jnp.zeros((N, D_HID), jnp.float32)
    z128 = jnp.zeros((N, D_OUT), jnp.float32)
    b1r = jnp.broadcast_to(b1, (8, D_HID))
    b2r = jnp.broadcast_to(b2, (8, D_OUT))

    deg_part = _sc_degree(edge_index[1], edge_attr, z1)
    dinv = _tc_dinv(deg_part[0].reshape(N_PAD // 128, 128),
                    deg_part[1].reshape(N_PAD // 128, 128)).reshape(N_PAD)

    h1 = _tc_matmul(x, W1)
    acc1, norm = _sc_layer1(src_all, dst_all, ew_all, dinv, h1, z64)
    h2 = _tc_combine_matmul(acc1[0], acc1[1], b1r, W2)
    acc2 = _sc_layer2(src_all, dst_all, norm, h2, z128)
    out = _tc_final(acc2[0], acc2[1], b2r)
    return out
